# baseline (speedup 1.0000x reference)
import jax
import jax.numpy as jnp
from jax import lax
from jax.experimental import pallas as pl
from jax.experimental.pallas import tpu as pltpu
from jax.experimental.pallas import tpu_sc as plsc

N_NODES = 50000
N_EDGES = 800000
BLK = 1024
N_PAD = 50176
GRID = N_PAD // BLK
TILE_ROWS = N_PAD // 16

NSC = 2
NTILE = 16

A_EDGES_PER_SC = N_EDGES // NSC
A_EDGES_PER_TILE = A_EDGES_PER_SC // NTILE
A_CHUNK = 5000

B_EDGES_PER_TILE = N_EDGES // NTILE
B_CHUNK = 2000
QW = 16


def _mesh():
    return plsc.VectorSubcoreMesh(core_axis_name="c", subcore_axis_name="s")


_SC_PARAMS = pltpu.CompilerParams(use_tc_tiling_on_sc=False)


def _count_body(ei_hbm, zeros_hbm, ones_hbm, c0_hbm, c1_hbm,
                idx_v, ones_v, counts_sp):
    c = lax.axis_index("c")
    s = lax.axis_index("s")
    row0 = pl.multiple_of(s * TILE_ROWS, 8)
    pltpu.sync_copy(zeros_hbm.at[pl.ds(row0, TILE_ROWS)],
                    counts_sp.at[pl.ds(row0, TILE_ROWS)])
    pltpu.sync_copy(ones_hbm, ones_v)
    plsc.subcore_barrier()
    base_t = N_EDGES + (c * NTILE + s) * A_EDGES_PER_TILE
    for k in range(A_EDGES_PER_TILE // A_CHUNK):
        base = pl.multiple_of(base_t + k * A_CHUNK, 8)
        pltpu.sync_copy(ei_hbm.at[pl.ds(base, A_CHUNK)], idx_v)
        pltpu.sync_copy(ones_v, counts_sp.at[idx_v], add=True)
    plsc.subcore_barrier()

    @pl.when(c == 0)
    def _():
        pltpu.sync_copy(counts_sp.at[pl.ds(row0, TILE_ROWS)],
                        c0_hbm.at[pl.ds(row0, TILE_ROWS)])

    @pl.when(c == 1)
    def _():
        pltpu.sync_copy(counts_sp.at[pl.ds(row0, TILE_ROWS)],
                        c1_hbm.at[pl.ds(row0, TILE_ROWS)])


def _sc_count(eflat, zeros_1d, ones_1d):
    f = pl.kernel(
        _count_body,
        mesh=_mesh(),
        out_type=[jax.ShapeDtypeStruct((N_PAD,), jnp.float32),
                  jax.ShapeDtypeStruct((N_PAD,), jnp.float32)],
        scratch_types=[pltpu.VMEM((A_CHUNK,), jnp.int32),
                       pltpu.VMEM((A_CHUNK,), jnp.float32),
                       pltpu.VMEM_SHARED((N_PAD,), jnp.float32)],
        compiler_params=_SC_PARAMS,
    )
    return f(eflat, zeros_1d, ones_1d)


def _edge_body(ei_hbm, yq0_hbm, yq1_hbm, yq2_hbm, yq3_hbm,
               s_hbm, sidx0, sidx1, didx0, didx1, rows0, rows1, acc_sp,
               gsem0, gsem1, ssem0, ssem1):
    c = lax.axis_index("c")
    s = lax.axis_index("s")
    row0 = pl.multiple_of(s * TILE_ROWS, 8)
    base_t = s * B_EDGES_PER_TILE
    ysc = ((yq0_hbm, yq1_hbm), (yq2_hbm, yq3_hbm))
    sidx = (sidx0, sidx1)
    didx = (didx0, didx1)
    rows = (rows0, rows1)
    gsems = (gsem0, gsem1)
    ssems = (ssem0, ssem1)
    nch = B_EDGES_PER_TILE // B_CHUNK
    for p in range(2):
        @pl.when(c == 0)
        def _():
            pltpu.sync_copy(ysc[0][p].at[pl.ds(row0, TILE_ROWS), :],
                            acc_sp.at[pl.ds(row0, TILE_ROWS), :])

        @pl.when(c == 1)
        def _():
            pltpu.sync_copy(ysc[1][p].at[pl.ds(row0, TILE_ROWS), :],
                            acc_sp.at[pl.ds(row0, TILE_ROWS), :])

        plsc.subcore_barrier()

        def _load_and_start(k, b, p=p):
            base = pl.multiple_of(base_t + k * B_CHUNK, 8)
            pltpu.sync_copy(ei_hbm.at[pl.ds(base, B_CHUNK)], sidx[b])
            pltpu.sync_copy(ei_hbm.at[pl.ds(N_EDGES + base, B_CHUNK)],
                            didx[b])

            @pl.when(c == 0)
            def _():
                pltpu.make_async_copy(ysc[0][p].at[sidx[b]], rows[b],
                                      gsems[b]).start()

            @pl.when(c == 1)
            def _():
                pltpu.make_async_copy(ysc[1][p].at[sidx[b]], rows[b],
                                      gsems[b]).start()

        _load_and_start(0, 0)
        for k in range(nch):
            b = k % 2
            pltpu.make_async_copy(ysc[0][p].at[sidx[b]], rows[b],
                                  gsems[b]).wait()
            pltpu.async_copy(rows[b], acc_sp.at[didx[b]], ssems[b], add=True)
            if k + 1 < nch:
                if k >= 1:
                    pltpu.make_async_copy(rows[1 - b],
                                          acc_sp.at[didx[1 - b]],
                                          ssems[1 - b]).wait()
                _load_and_start(k + 1, 1 - b)
        pltpu.make_async_copy(rows[(nch - 2) % 2],
                              acc_sp.at[didx[(nch - 2) % 2]],
                              ssems[(nch - 2) % 2]).wait()
        pltpu.make_async_copy(rows[(nch - 1) % 2],
                              acc_sp.at[didx[(nch - 1) % 2]],
                              ssems[(nch - 1) % 2]).wait()
        plsc.subcore_barrier()
        pltpu.sync_copy(acc_sp.at[pl.ds(row0, TILE_ROWS), :],
                        s_hbm.at[2 * c + p, pl.ds(row0, TILE_ROWS), :])


def _sc_edges(eflat, yq0, yq1, yq2, yq3):
    f = pl.kernel(
        _edge_body,
        mesh=_mesh(),
        out_type=jax.ShapeDtypeStruct((4, N_PAD, QW), jnp.float32),
        scratch_types=[pltpu.VMEM((B_CHUNK,), jnp.int32),
                       pltpu.VMEM((B_CHUNK,), jnp.int32),
                       pltpu.VMEM((B_CHUNK,), jnp.int32),
                       pltpu.VMEM((B_CHUNK,), jnp.int32),
                       pltpu.VMEM((B_CHUNK, QW), jnp.float32),
                       pltpu.VMEM((B_CHUNK, QW), jnp.float32),
                       pltpu.VMEM_SHARED((N_PAD, QW), jnp.float32),
                       pltpu.SemaphoreType.DMA,
                       pltpu.SemaphoreType.DMA,
                       pltpu.SemaphoreType.DMA,
                       pltpu.SemaphoreType.DMA],
        compiler_params=_SC_PARAMS,
    )
    return f(eflat, yq0, yq1, yq2, yq3)


def _front_body(x_ref, w1_ref, b1_ref, wg_ref, dinv_ref,
                y0_ref, y1_ref, y2_ref, y3_ref):
    h = jnp.maximum(
        jnp.dot(x_ref[...], w1_ref[...], preferred_element_type=jnp.float32)
        + b1_ref[...], 0.0)
    xw = jnp.dot(h, wg_ref[...], preferred_element_type=jnp.float32)
    y = xw * dinv_ref[...]
    y4 = y.reshape(BLK // 8, 8, 64)
    for q, ref in enumerate((y0_ref, y1_ref, y2_ref, y3_ref)):
        ref[...] = y4[:, :, q * QW:(q + 1) * QW].reshape(BLK // 8, 128)


def _tc_front(x_pad, W1, b1r, Wg, dinv):
    return pl.pallas_call(
        _front_body,
        grid=(GRID,),
        in_specs=[pl.BlockSpec((BLK, 10), lambda i: (i, 0)),
                  pl.BlockSpec((10, 64), lambda i: (0, 0)),
                  pl.BlockSpec((1, 64), lambda i: (0, 0)),
                  pl.BlockSpec((64, 64), lambda i: (0, 0)),
                  pl.BlockSpec((BLK, 1), lambda i: (i, 0))],
        out_specs=[pl.BlockSpec((BLK // 8, 128), lambda i: (i, 0))] * 4,
        out_shape=[jax.ShapeDtypeStruct((N_PAD // 8, 128), jnp.float32)] * 4,
    )(x_pad, W1, b1r, Wg, dinv)


def _post_body(s_ref, dinv_ref, bg_ref, w2bd_ref, b2t_ref, out_ref):
    dinv = dinv_ref[...]
    dp = jnp.broadcast_to(dinv.reshape(BLK // 8, 8, 1),
                          (BLK // 8, 8, QW)).reshape(BLK // 8, 128)
    total = jnp.broadcast_to(b2t_ref[...], (BLK // 8, 128))
    for q in range(4):
        bgq = bg_ref[0:1, q * QW:(q + 1) * QW]
        bgp = jnp.concatenate([bgq] * 8, axis=1)
        t = jnp.maximum(dp * s_ref[q] + bgp, 0.0)
        total = total + jnp.dot(t, w2bd_ref[q],
                                preferred_element_type=jnp.float32)
    out_ref[...] = total


def _tc_post(S_t, dinv, bgr, W2bd, b2t):
    return pl.pallas_call(
        _post_body,
        grid=(GRID,),
        in_specs=[pl.BlockSpec((4, BLK // 8, 128), lambda i: (0, i, 0)),
                  pl.BlockSpec((BLK, 1), lambda i: (i, 0)),
                  pl.BlockSpec((1, 64), lambda i: (0, 0)),
                  pl.BlockSpec((4, 128, 128), lambda i: (0, 0, 0)),
                  pl.BlockSpec((1, 128), lambda i: (0, 0))],
        out_specs=pl.BlockSpec((BLK // 8, 128), lambda i: (i, 0)),
        out_shape=jax.ShapeDtypeStruct((N_PAD // 8, 128), jnp.float32),
    )(S_t, dinv, bgr, W2bd, b2t)


@jax.jit
def kernel(x, edge_index, W1, b1, Wg, bg, W2, b2):
    eflat = edge_index.astype(jnp.int32).reshape(2 * N_EDGES)

    x_pad = jnp.zeros((N_PAD, 10), jnp.float32).at[:N_NODES].set(x)
    zeros_1d = jnp.zeros((N_PAD,), jnp.float32)
    ones_1d = jnp.ones((A_CHUNK,), jnp.float32)
    c0, c1 = _sc_count(eflat, zeros_1d, ones_1d)
    dinv = lax.rsqrt(c0 + c1 + 1.0).reshape(N_PAD, 1)
    y0, y1, y2, y3 = _tc_front(x_pad, W1, b1.reshape(1, 64), Wg, dinv)
    yqs = [y.reshape(N_PAD, QW) for y in (y0, y1, y2, y3)]
    S = _sc_edges(eflat, *yqs)
    S_t = S.reshape(4, N_PAD // 8, 128)
    W2r = W2.reshape(4, QW, 10)
    eye8 = jnp.eye(8, dtype=jnp.float32)
    W2bd = (eye8[None, :, None, :, None]
            * W2r[:, None, :, None, :]).reshape(4, 128, 80)
    W2bd = jnp.pad(W2bd, ((0, 0), (0, 0), (0, 48)))
    b2t = jnp.pad(jnp.tile(b2, 8), (0, 48)).reshape(1, 128)
    out = _tc_post(S_t, dinv, bg.reshape(1, 64), W2bd, b2t)
    return out[:, :80].reshape(N_PAD, 10)[:N_NODES]


# speedup vs baseline: 1.1722x; 1.1722x over previous
import jax
import jax.numpy as jnp
from jax import lax
from jax.experimental import pallas as pl
from jax.experimental.pallas import tpu as pltpu
from jax.experimental.pallas import tpu_sc as plsc

N_NODES = 50000
N_EDGES = 800000
BLK = 1024
N_PAD = 50176
GRID = N_PAD // BLK
TILE_ROWS = N_PAD // 16

NSC = 2
NTILE = 16

A_EDGES_PER_SC = N_EDGES // NSC
A_EDGES_PER_TILE = A_EDGES_PER_SC // NTILE
A_CHUNK = 5000

B_EDGES_PER_TILE = N_EDGES // NTILE
B_CHUNK = 2000
QW = 16


def _mesh():
    return plsc.VectorSubcoreMesh(core_axis_name="c", subcore_axis_name="s")


_SC_PARAMS = pltpu.CompilerParams(use_tc_tiling_on_sc=False)


def _count_body(ei_hbm, zeros_hbm, ones_hbm, c0_hbm, c1_hbm,
                idx_v, ones_v, counts_sp):
    c = lax.axis_index("c")
    s = lax.axis_index("s")
    row0 = pl.multiple_of(s * TILE_ROWS, 8)
    pltpu.sync_copy(zeros_hbm.at[pl.ds(row0, TILE_ROWS)],
                    counts_sp.at[pl.ds(row0, TILE_ROWS)])
    pltpu.sync_copy(ones_hbm, ones_v)
    plsc.subcore_barrier()
    base_t = N_EDGES + (c * NTILE + s) * A_EDGES_PER_TILE
    for k in range(A_EDGES_PER_TILE // A_CHUNK):
        base = pl.multiple_of(base_t + k * A_CHUNK, 8)
        pltpu.sync_copy(ei_hbm.at[pl.ds(base, A_CHUNK)], idx_v)
        pltpu.sync_copy(ones_v, counts_sp.at[idx_v], add=True)
    plsc.subcore_barrier()

    @pl.when(c == 0)
    def _():
        pltpu.sync_copy(counts_sp.at[pl.ds(row0, TILE_ROWS)],
                        c0_hbm.at[pl.ds(row0, TILE_ROWS)])

    @pl.when(c == 1)
    def _():
        pltpu.sync_copy(counts_sp.at[pl.ds(row0, TILE_ROWS)],
                        c1_hbm.at[pl.ds(row0, TILE_ROWS)])


def _sc_count(eflat, zeros_1d, ones_1d):
    f = pl.kernel(
        _count_body,
        mesh=_mesh(),
        out_type=[jax.ShapeDtypeStruct((N_PAD,), jnp.float32),
                  jax.ShapeDtypeStruct((N_PAD,), jnp.float32)],
        scratch_types=[pltpu.VMEM((A_CHUNK,), jnp.int32),
                       pltpu.VMEM((A_CHUNK,), jnp.float32),
                       pltpu.VMEM_SHARED((N_PAD,), jnp.float32)],
        compiler_params=_SC_PARAMS,
    )
    return f(eflat, zeros_1d, ones_1d)


def _edge_body(ei_hbm, yq0_hbm, yq1_hbm, yq2_hbm, yq3_hbm,
               s_hbm, sidx0, sidx1, didx0, didx1, rows0, rows1, acc_sp,
               sem0, sem1):
    c = lax.axis_index("c")
    s = lax.axis_index("s")
    row0 = pl.multiple_of(s * TILE_ROWS, 8)
    base_t = s * B_EDGES_PER_TILE
    ysc = ((yq0_hbm, yq1_hbm), (yq2_hbm, yq3_hbm))
    sidx = (sidx0, sidx1)
    didx = (didx0, didx1)
    rows = (rows0, rows1)
    sems = (sem0, sem1)
    nch = B_EDGES_PER_TILE // B_CHUNK
    for p in range(2):
        @pl.when(c == 0)
        def _():
            pltpu.sync_copy(ysc[0][p].at[pl.ds(row0, TILE_ROWS), :],
                            acc_sp.at[pl.ds(row0, TILE_ROWS), :])

        @pl.when(c == 1)
        def _():
            pltpu.sync_copy(ysc[1][p].at[pl.ds(row0, TILE_ROWS), :],
                            acc_sp.at[pl.ds(row0, TILE_ROWS), :])

        plsc.subcore_barrier()

        def _load_and_start(k, b, p=p):
            base = pl.multiple_of(base_t + k * B_CHUNK, 8)
            pltpu.sync_copy(ei_hbm.at[pl.ds(base, B_CHUNK)], sidx[b])
            pltpu.sync_copy(ei_hbm.at[pl.ds(N_EDGES + base, B_CHUNK)],
                            didx[b])

            @pl.when(c == 0)
            def _():
                pltpu.make_async_copy(ysc[0][p].at[sidx[b]], rows[b],
                                      sems[b]).start()

            @pl.when(c == 1)
            def _():
                pltpu.make_async_copy(ysc[1][p].at[sidx[b]], rows[b],
                                      sems[b]).start()

        _load_and_start(0, 0)
        for k in range(nch):
            b = k % 2
            if k + 1 < nch:
                _load_and_start(k + 1, (k + 1) % 2)
            pltpu.make_async_copy(ysc[0][p].at[sidx[b]], rows[b],
                                  sems[b]).wait()
            pltpu.sync_copy(rows[b], acc_sp.at[didx[b]], add=True)
        plsc.subcore_barrier()
        pltpu.sync_copy(acc_sp.at[pl.ds(row0, TILE_ROWS), :],
                        s_hbm.at[2 * c + p, pl.ds(row0, TILE_ROWS), :])


def _sc_edges(eflat, yq0, yq1, yq2, yq3):
    f = pl.kernel(
        _edge_body,
        mesh=_mesh(),
        out_type=jax.ShapeDtypeStruct((4, N_PAD, QW), jnp.float32),
        scratch_types=[pltpu.VMEM((B_CHUNK,), jnp.int32),
                       pltpu.VMEM((B_CHUNK,), jnp.int32),
                       pltpu.VMEM((B_CHUNK,), jnp.int32),
                       pltpu.VMEM((B_CHUNK,), jnp.int32),
                       pltpu.VMEM((B_CHUNK, QW), jnp.float32),
                       pltpu.VMEM((B_CHUNK, QW), jnp.float32),
                       pltpu.VMEM_SHARED((N_PAD, QW), jnp.float32),
                       pltpu.SemaphoreType.DMA,
                       pltpu.SemaphoreType.DMA],
        compiler_params=_SC_PARAMS,
    )
    return f(eflat, yq0, yq1, yq2, yq3)


def _front_body(x_ref, w1_ref, b1_ref, wg_ref, dinv_ref,
                y0_ref, y1_ref, y2_ref, y3_ref):
    h = jnp.maximum(
        jnp.dot(x_ref[...], w1_ref[...], preferred_element_type=jnp.float32)
        + b1_ref[...], 0.0)
    xw = jnp.dot(h, wg_ref[...], preferred_element_type=jnp.float32)
    y = xw * dinv_ref[...]
    y4 = y.reshape(BLK // 8, 8, 64)
    for q, ref in enumerate((y0_ref, y1_ref, y2_ref, y3_ref)):
        ref[...] = y4[:, :, q * QW:(q + 1) * QW].reshape(BLK // 8, 128)


def _tc_front(x_pad, W1, b1r, Wg, dinv):
    return pl.pallas_call(
        _front_body,
        grid=(GRID,),
        in_specs=[pl.BlockSpec((BLK, 10), lambda i: (i, 0)),
                  pl.BlockSpec((10, 64), lambda i: (0, 0)),
                  pl.BlockSpec((1, 64), lambda i: (0, 0)),
                  pl.BlockSpec((64, 64), lambda i: (0, 0)),
                  pl.BlockSpec((BLK, 1), lambda i: (i, 0))],
        out_specs=[pl.BlockSpec((BLK // 8, 128), lambda i: (i, 0))] * 4,
        out_shape=[jax.ShapeDtypeStruct((N_PAD // 8, 128), jnp.float32)] * 4,
    )(x_pad, W1, b1r, Wg, dinv)


def _post_body(s_ref, dinv_ref, bg_ref, w2bd_ref, b2t_ref, out_ref):
    dinv = dinv_ref[...]
    dp = jnp.broadcast_to(dinv.reshape(BLK // 8, 8, 1),
                          (BLK // 8, 8, QW)).reshape(BLK // 8, 128)
    total = jnp.broadcast_to(b2t_ref[...], (BLK // 8, 128))
    for q in range(4):
        bgq = bg_ref[0:1, q * QW:(q + 1) * QW]
        bgp = jnp.concatenate([bgq] * 8, axis=1)
        t = jnp.maximum(dp * s_ref[q] + bgp, 0.0)
        total = total + jnp.dot(t, w2bd_ref[q],
                                preferred_element_type=jnp.float32)
    out_ref[...] = total


def _tc_post(S_t, dinv, bgr, W2bd, b2t):
    return pl.pallas_call(
        _post_body,
        grid=(GRID,),
        in_specs=[pl.BlockSpec((4, BLK // 8, 128), lambda i: (0, i, 0)),
                  pl.BlockSpec((BLK, 1), lambda i: (i, 0)),
                  pl.BlockSpec((1, 64), lambda i: (0, 0)),
                  pl.BlockSpec((4, 128, 128), lambda i: (0, 0, 0)),
                  pl.BlockSpec((1, 128), lambda i: (0, 0))],
        out_specs=pl.BlockSpec((BLK // 8, 128), lambda i: (i, 0)),
        out_shape=jax.ShapeDtypeStruct((N_PAD // 8, 128), jnp.float32),
    )(S_t, dinv, bgr, W2bd, b2t)


@jax.jit
def kernel(x, edge_index, W1, b1, Wg, bg, W2, b2):
    eflat = edge_index.astype(jnp.int32).reshape(2 * N_EDGES)

    x_pad = jnp.zeros((N_PAD, 10), jnp.float32).at[:N_NODES].set(x)
    zeros_1d = jnp.zeros((N_PAD,), jnp.float32)
    ones_1d = jnp.ones((A_CHUNK,), jnp.float32)
    c0, c1 = _sc_count(eflat, zeros_1d, ones_1d)
    dinv = lax.rsqrt(c0 + c1 + 1.0).reshape(N_PAD, 1)
    y0, y1, y2, y3 = _tc_front(x_pad, W1, b1.reshape(1, 64), Wg, dinv)
    yqs = [y.reshape(N_PAD, QW) for y in (y0, y1, y2, y3)]
    S = _sc_edges(eflat, *yqs)
    S_t = S.reshape(4, N_PAD // 8, 128)
    W2r = W2.reshape(4, QW, 10)
    eye8 = jnp.eye(8, dtype=jnp.float32)
    W2bd = (eye8[None, :, None, :, None]
            * W2r[:, None, :, None, :]).reshape(4, 128, 80)
    W2bd = jnp.pad(W2bd, ((0, 0), (0, 0), (0, 48)))
    b2t = jnp.pad(jnp.tile(b2, 8), (0, 48)).reshape(1, 128)
    out = _tc_post(S_t, dinv, bg.reshape(1, 64), W2bd, b2t)
    return out[:, :80].reshape(N_PAD, 10)[:N_NODES]
